# baseline (speedup 1.0000x reference)
import functools

import jax
import jax.numpy as jnp
from jax import lax
from jax.experimental import pallas as pl
from jax.experimental.pallas import tpu as pltpu
from jax.experimental.pallas import tpu_sc as plsc

N_NODES = 10000
D = 128
DW = D // 2
NUM_CHANNELS = 2
NTILES = 16
NP = 10240
NN = 10000
ROWS_PER_TILE = NN // NTILES
EB = 128
K = 2


def _sc_agg_build(e_pad):
    e_per_tile = e_pad // NTILES
    blocks_per_tile = e_per_tile // EB
    n_super = blocks_per_tile // K
    ew_rows_per_channel = e_pad // EB
    mesh = plsc.VectorSubcoreMesh(core_axis_name="c", subcore_axis_name="s")

    @functools.partial(
        pl.kernel,
        mesh=mesh,
        out_type=jax.ShapeDtypeStruct((NUM_CHANNELS, NN, D), jnp.float32),
        compiler_params=pltpu.CompilerParams(use_tc_tiling_on_sc=False),
        scratch_types=[
            pltpu.VMEM((K, EB), jnp.int32),
            pltpu.VMEM((K, EB), jnp.int32),
            pltpu.VMEM((K, EB), jnp.float32),
        ]
        + [pltpu.VMEM((EB, DW), jnp.int32)] * K
        + [pltpu.VMEM((EB, D), jnp.float32)] * K
        + [
            pltpu.VMEM_SHARED((NN, D), jnp.float32),
        ]
        + [pltpu.SemaphoreType.DMA] * (2 * K),
    )
    def sc_agg(node_hbm, src_hbm, dst_hbm, ew_hbm, out_hbm,
               src_v, dst_v, ew_v, *rest):
        rows_p = rest[:K]
        rows_f = rest[K:2 * K]
        acc_s = rest[2 * K]
        gsem = rest[2 * K + 1:2 * K + 1 + K]
        ssem = rest[2 * K + 1 + K:]
        cid = lax.axis_index("c")
        sid = lax.axis_index("s")

        zeros16 = jnp.zeros((16,), jnp.float32)

        def zero_row(i, _):
            for j in range(D // 16):
                rows_f[0][i, pl.ds(j * 16, 16)] = zeros16
            return 0

        lax.fori_loop(0, EB, zero_row, 0)
        for k in range(5):
            pltpu.sync_copy(
                rows_f[0].at[pl.ds(0, 125)],
                acc_s.at[pl.ds(sid * ROWS_PER_TILE + k * 125, 125)])
        plsc.subcore_barrier()

        blk_base = sid * blocks_per_tile
        himask = jnp.full((16,), -65536, jnp.int32)

        def super_body(g, _):
            blk = blk_base + g * K
            pltpu.sync_copy(src_hbm.at[pl.ds(blk, K)], src_v)
            pltpu.sync_copy(dst_hbm.at[pl.ds(blk, K)], dst_v)
            pltpu.sync_copy(
                ew_hbm.at[pl.ds(cid * ew_rows_per_channel + blk, K)], ew_v)
            gathers = [
                pltpu.async_copy(node_hbm.at[src_v.at[j]], rows_p[j],
                                 gsem[j])
                for j in range(K)
            ]
            scatters = []
            for j in range(K):
                gathers[j].wait()

                def scale_group(grp, _):
                    ew16 = ew_v[j, pl.ds(grp * 16, 16)]
                    for l in range(16):
                        wv = ew16.at[jnp.full((16,), l, jnp.int32)].get(
                            mode="promise_in_bounds")
                        e = grp * 16 + l
                        for f in range(DW // 16):
                            u = rows_p[j][e, pl.ds(f * 16, 16)]
                            lo = lax.bitcast_convert_type(u << 16,
                                                          jnp.float32)
                            hi = lax.bitcast_convert_type(u & himask,
                                                          jnp.float32)
                            rows_f[j][e, pl.ds(f * 32, 16)] = lo * wv
                            rows_f[j][e, pl.ds(f * 32 + 16, 16)] = hi * wv
                    return 0

                lax.fori_loop(0, EB // 16, scale_group, 0)
                scatters.append(
                    pltpu.async_copy(rows_f[j], acc_s.at[dst_v.at[j]],
                                     ssem[j], add=True))
            for s in scatters:
                s.wait()
            return 0

        lax.fori_loop(0, n_super, super_body, 0)
        plsc.subcore_barrier()

        pltpu.sync_copy(
            acc_s.at[pl.ds(sid * ROWS_PER_TILE, ROWS_PER_TILE)],
            out_hbm.at[cid, pl.ds(sid * ROWS_PER_TILE, ROWS_PER_TILE)])

    return sc_agg


def _dense_body(a0, a1, w0, w1, b0, b1, wf, bf, o):
    h0 = jnp.maximum(
        jnp.dot(a0[...], w0[...], preferred_element_type=jnp.float32)
        + b0[...], 0.0)
    h1 = jnp.maximum(
        jnp.dot(a1[...], w1[...], preferred_element_type=jnp.float32)
        + b1[...], 0.0)
    o[...] = jnp.dot(h0 + h1, wf[...],
                     preferred_element_type=jnp.float32) + bf[...]


RB = 400

_tc_dense = pl.pallas_call(
    _dense_body,
    grid=(NN // RB,),
    in_specs=[
        pl.BlockSpec((RB, D), lambda i: (i, 0)),
        pl.BlockSpec((RB, D), lambda i: (i, 0)),
        pl.BlockSpec((D, D), lambda i: (0, 0)),
        pl.BlockSpec((D, D), lambda i: (0, 0)),
        pl.BlockSpec((1, D), lambda i: (0, 0)),
        pl.BlockSpec((1, D), lambda i: (0, 0)),
        pl.BlockSpec((D, D), lambda i: (0, 0)),
        pl.BlockSpec((1, D), lambda i: (0, 0)),
    ],
    out_specs=pl.BlockSpec((RB, D), lambda i: (i, 0)),
    out_shape=jax.ShapeDtypeStruct((NN, D), jnp.float32),
)


def kernel(node_state, edge_index, edge_weight, weight, bias, final_W, final_b):
    e = edge_weight.shape[0]
    unit = NTILES * EB * K
    e_pad = ((e + unit - 1) // unit) * unit
    pad = e_pad - e
    src = jnp.pad(edge_index[0].astype(jnp.int32), (0, pad)).reshape(-1, EB)
    dst = jnp.pad(edge_index[1].astype(jnp.int32), (0, pad)).reshape(-1, EB)
    ewt = jnp.pad(edge_weight.astype(jnp.float32).T,
                  ((0, 0), (0, pad))).reshape(-1, EB)
    n = node_state.shape[0]
    nb = (node_state.reshape(n, D // 32, 2, 16)
          .transpose(0, 1, 3, 2).reshape(n, DW, 2).astype(jnp.bfloat16))
    nbi = lax.bitcast_convert_type(nb, jnp.int32)
    nbi = jnp.pad(nbi, ((0, NP - n), (0, 0)))

    agg = _sc_agg_build(e_pad)(nbi, src, dst, ewt)

    out = _tc_dense(agg[0], agg[1],
                    weight[:, :, 0], weight[:, :, 1],
                    bias[:, 0][None, :], bias[:, 1][None, :],
                    final_W.T, final_b[None, :])
    return out[:N_NODES]


# speedup vs baseline: 1.4539x; 1.4539x over previous
import functools

import jax
import jax.numpy as jnp
from jax import lax
from jax.experimental import pallas as pl
from jax.experimental.pallas import tpu as pltpu
from jax.experimental.pallas import tpu_sc as plsc

N_NODES = 10000
D = 128
DW = D // 2
NUM_CHANNELS = 2
NTILES = 16
NP = 10240
NN = 10000
ROWS_PER_TILE = NN // NTILES
EB = 96
K = 4
NSB = 2


def _sc_agg_build(e_pad):
    e_per_tile = e_pad // NTILES
    blocks_per_tile = e_per_tile // EB
    n_super = blocks_per_tile // K
    ew_rows_per_channel = e_pad // EB
    mesh = plsc.VectorSubcoreMesh(core_axis_name="c", subcore_axis_name="s")

    @functools.partial(
        pl.kernel,
        mesh=mesh,
        out_type=jax.ShapeDtypeStruct((NUM_CHANNELS, NN, D), jnp.float32),
        compiler_params=pltpu.CompilerParams(use_tc_tiling_on_sc=False),
        scratch_types=[
            pltpu.VMEM((K, EB), jnp.int32),
            pltpu.VMEM((K, EB), jnp.int32),
            pltpu.VMEM((K, EB), jnp.float32),
        ]
        + [pltpu.VMEM((EB, DW), jnp.int32)] * K
        + [pltpu.VMEM((EB, D), jnp.float32)] * NSB
        + [
            pltpu.VMEM_SHARED((NN, D), jnp.float32),
        ]
        + [pltpu.SemaphoreType.DMA] * (K + NSB),
    )
    def sc_agg(node_hbm, src_hbm, dst_hbm, ew_hbm, out_hbm,
               src_v, dst_v, ew_v, *rest):
        rows_p = rest[:K]
        rows_f = rest[K:K + NSB]
        acc_s = rest[K + NSB]
        gsem = rest[K + NSB + 1:K + NSB + 1 + K]
        ssem = rest[K + NSB + 1 + K:]
        cid = lax.axis_index("c")
        sid = lax.axis_index("s")

        zeros16 = jnp.zeros((16,), jnp.float32)

        def zero_row(i, _):
            for j in range(D // 16):
                rows_f[0][i, pl.ds(j * 16, 16)] = zeros16
            return 0

        lax.fori_loop(0, EB, zero_row, 0)
        nz = (ROWS_PER_TILE + EB - 1) // EB
        for k in range(nz):
            w = min(EB, ROWS_PER_TILE - k * EB)
            pltpu.sync_copy(
                rows_f[0].at[pl.ds(0, w)],
                acc_s.at[pl.ds(sid * ROWS_PER_TILE + k * EB, w)])
        plsc.subcore_barrier()

        blk_base = sid * blocks_per_tile
        himask = jnp.full((16,), -65536, jnp.int32)

        def super_body(g, _):
            blk = blk_base + g * K
            pltpu.sync_copy(src_hbm.at[pl.ds(blk, K)], src_v)
            pltpu.sync_copy(dst_hbm.at[pl.ds(blk, K)], dst_v)
            pltpu.sync_copy(
                ew_hbm.at[pl.ds(cid * ew_rows_per_channel + blk, K)], ew_v)
            gathers = [
                pltpu.async_copy(node_hbm.at[src_v.at[j]], rows_p[j],
                                 gsem[j])
                for j in range(K)
            ]
            scatters = []
            for j in range(K):
                b = j % NSB
                if j >= NSB:
                    scatters[j - NSB].wait()
                gathers[j].wait()

                def scale_group(grp, _):
                    ew16 = ew_v[j, pl.ds(grp * 16, 16)]
                    for l in range(16):
                        wv = ew16.at[jnp.full((16,), l, jnp.int32)].get(
                            mode="promise_in_bounds")
                        e = grp * 16 + l
                        us = [rows_p[j][e, pl.ds(f * 16, 16)]
                              for f in range(DW // 16)]
                        los = [lax.bitcast_convert_type(u << 16, jnp.float32)
                               for u in us]
                        his = [lax.bitcast_convert_type(u & himask,
                                                        jnp.float32)
                               for u in us]
                        for f in range(DW // 16):
                            rows_f[b][e, pl.ds(f * 32, 16)] = los[f] * wv
                            rows_f[b][e, pl.ds(f * 32 + 16, 16)] = (
                                his[f] * wv)
                    return 0

                lax.fori_loop(0, EB // 16, scale_group, 0)
                scatters.append(
                    pltpu.async_copy(rows_f[b], acc_s.at[dst_v.at[j]],
                                     ssem[b], add=True))
            for s in scatters[K - NSB:]:
                s.wait()
            return 0

        lax.fori_loop(0, n_super, super_body, 0)
        plsc.subcore_barrier()

        pltpu.sync_copy(
            acc_s.at[pl.ds(sid * ROWS_PER_TILE, ROWS_PER_TILE)],
            out_hbm.at[cid, pl.ds(sid * ROWS_PER_TILE, ROWS_PER_TILE)])

    return sc_agg


def _dense_body(a0, a1, w0, w1, b0, b1, wf, bf, o):
    h0 = jnp.maximum(
        jnp.dot(a0[...], w0[...], preferred_element_type=jnp.float32)
        + b0[...], 0.0)
    h1 = jnp.maximum(
        jnp.dot(a1[...], w1[...], preferred_element_type=jnp.float32)
        + b1[...], 0.0)
    o[...] = jnp.dot(h0 + h1, wf[...],
                     preferred_element_type=jnp.float32) + bf[...]


RB = 400

_tc_dense = pl.pallas_call(
    _dense_body,
    grid=(NN // RB,),
    in_specs=[
        pl.BlockSpec((RB, D), lambda i: (i, 0)),
        pl.BlockSpec((RB, D), lambda i: (i, 0)),
        pl.BlockSpec((D, D), lambda i: (0, 0)),
        pl.BlockSpec((D, D), lambda i: (0, 0)),
        pl.BlockSpec((1, D), lambda i: (0, 0)),
        pl.BlockSpec((1, D), lambda i: (0, 0)),
        pl.BlockSpec((D, D), lambda i: (0, 0)),
        pl.BlockSpec((1, D), lambda i: (0, 0)),
    ],
    out_specs=pl.BlockSpec((RB, D), lambda i: (i, 0)),
    out_shape=jax.ShapeDtypeStruct((NN, D), jnp.float32),
)


def kernel(node_state, edge_index, edge_weight, weight, bias, final_W, final_b):
    e = edge_weight.shape[0]
    unit = NTILES * EB * K
    e_pad = ((e + unit - 1) // unit) * unit
    pad = e_pad - e
    src = jnp.pad(edge_index[0].astype(jnp.int32), (0, pad)).reshape(-1, EB)
    dst = jnp.pad(edge_index[1].astype(jnp.int32), (0, pad)).reshape(-1, EB)
    ewt = jnp.pad(edge_weight.astype(jnp.float32).T,
                  ((0, 0), (0, pad))).reshape(-1, EB)
    n = node_state.shape[0]
    nb = (node_state.reshape(n, D // 32, 2, 16)
          .transpose(0, 1, 3, 2).reshape(n, DW, 2).astype(jnp.bfloat16))
    nbi = lax.bitcast_convert_type(nb, jnp.int32)
    nbi = jnp.pad(nbi, ((0, NP - n), (0, 0)))

    agg = _sc_agg_build(e_pad)(nbi, src, dst, ewt)

    out = _tc_dense(agg[0], agg[1],
                    weight[:, :, 0], weight[:, :, 1],
                    bias[:, 0][None, :], bias[:, 1][None, :],
                    final_W.T, final_b[None, :])
    return out[:N_NODES]
